# baseline (speedup 1.0000x reference)
import jax
import jax.numpy as jnp
from jax import lax
from jax.experimental import pallas as pl
from jax.experimental.pallas import tpu as pltpu
from jax.experimental.pallas import tpu_sc as plsc

N_NODES = 10000
DIM = 128
HEADS = 8
NPAD = 10112
NWORK = 32
NB1 = 128
NB2 = 16
EP = 360448
NC1 = EP // (NWORK * NB1)
NC2 = EP // (NWORK * NB2)

_SC_PARAMS = pltpu.CompilerParams(use_tc_tiling_on_sc=False)
_SC_PARAMS2 = pltpu.CompilerParams(use_tc_tiling_on_sc=False,
                                   needs_layout_passes=False)


def _leaky(v):
    return jnp.where(v > 0, v, v * 0.2)


def _proj_body(x_ref, w_ref, attcat_ref, xp_ref, acat_ref, amax_ref):
    i = pl.program_id(0)
    x_blk = x_ref[...]
    xp_ref[...] = lax.dot_general(x_blk, w_ref[...], (((1,), (1,)), ((), ())),
                                  preferred_element_type=jnp.float32)
    rows = []
    for k in range(16):
        wk = w_ref[pl.ds((k % HEADS) * DIM, DIM), :]
        rows.append(jnp.dot(attcat_ref[k:k + 1, :], wk,
                            preferred_element_type=jnp.float32))
    vcat = jnp.concatenate(rows, axis=0)
    acat = lax.dot_general(x_blk, vcat, (((1,), (1,)), ((), ())),
                           preferred_element_type=jnp.float32)
    acat_ref[...] = acat
    cur = jnp.max(acat, axis=0, keepdims=True)

    @pl.when(i == 0)
    def _():
        amax_ref[...] = cur

    @pl.when(i > 0)
    def _():
        amax_ref[...] = jnp.maximum(amax_ref[...], cur)


def _project(x, W, att_cat):
    nblk = 10
    B = N_NODES // nblk
    return pl.pallas_call(
        _proj_body,
        grid=(nblk,),
        in_specs=[
            pl.BlockSpec((B, DIM), lambda i: (i, 0)),
            pl.BlockSpec((HEADS * DIM, DIM), lambda i: (0, 0)),
            pl.BlockSpec((16, DIM), lambda i: (0, 0)),
        ],
        out_specs=[
            pl.BlockSpec((B, HEADS * DIM), lambda i: (i, 0)),
            pl.BlockSpec((B, 16), lambda i: (i, 0)),
            pl.BlockSpec((1, 16), lambda i: (0, 0)),
        ],
        out_shape=[
            jax.ShapeDtypeStruct((N_NODES, HEADS * DIM), jnp.float32),
            jax.ShapeDtypeStruct((N_NODES, 16), jnp.float32),
            jax.ShapeDtypeStruct((1, 16), jnp.float32),
        ],
    )(x, W, att_cat)


def _sc_denom_body(src2_hbm, dst2_hbm, s_hbm, d_hbm, amax_hbm, zd_hbm,
                   denom_hbm, exf_hbm,
                   sall, dall, pa0, pa1, pb0, pb1, exb, amv, dsh,
                   semA0, semA1, semB0, semB1):
    c = lax.axis_index("c")
    s = lax.axis_index("s")
    w = s * 2 + c
    rows = NPAD // 16
    r0 = s * rows
    pltpu.sync_copy(zd_hbm.at[pl.ds(r0, rows)], dsh.at[pl.ds(r0, rows)])
    pltpu.sync_copy(amax_hbm, amv)
    pltpu.sync_copy(src2_hbm.at[pl.ds(w * NC1, NC1)], sall)
    pltpu.sync_copy(dst2_hbm.at[pl.ds(w * NC1, NC1)], dall)
    plsc.subcore_barrier()
    av = amv[...]
    pas = (pa0, pa1)
    pbs = (pb0, pb1)
    semAs = (semA0, semA1)
    semBs = (semB0, semB1)

    def fire(j, b):
        pltpu.async_copy(s_hbm.at[sall.at[j]], pas[b], semAs[b])
        pltpu.async_copy(d_hbm.at[dall.at[j]], pbs[b], semBs[b])

    fire(0, 0)

    def outer(i2, carry):
        for b in range(2):
            j = i2 * 2 + b
            pltpu.make_async_copy(s_hbm.at[sall.at[j]], pas[b], semAs[b]).wait()
            pltpu.make_async_copy(d_hbm.at[dall.at[j]], pbs[b], semBs[b]).wait()
            nxt = j + 1

            @pl.when(nxt < NC1)
            def _():
                fire(nxt, 1 - b)

            pa = pas[b]
            pb = pbs[b]

            @plsc.parallel_loop(0, NB1, 1, unroll=4)
            def _(e):
                va = pa[e]
                vb = pb[e]
                al = _leaky(va + vb)
                bd = _leaky(av + vb)
                exb[e] = jnp.exp(al - bd)
            pltpu.sync_copy(exb, dsh.at[dall.at[j]], add=True)
            gb = (w * NC1 + j) * NB1
            pltpu.sync_copy(exb, exf_hbm.at[pl.ds(gb, NB1)])
        return carry

    lax.fori_loop(0, NC1 // 2, outer, 0)
    plsc.subcore_barrier()
    pltpu.sync_copy(dsh.at[pl.ds(r0, rows)], denom_hbm.at[c, pl.ds(r0, rows)])


def _sc_denom(src2, dst2, S, D, amax16, zd):
    mesh = plsc.VectorSubcoreMesh(core_axis_name="c", subcore_axis_name="s")
    return pl.kernel(
        _sc_denom_body,
        out_type=[
            jax.ShapeDtypeStruct((2, NPAD, 16), jnp.float32),
            jax.ShapeDtypeStruct((EP, 16), jnp.float32),
        ],
        mesh=mesh,
        compiler_params=_SC_PARAMS,
        scratch_types=[
            pltpu.VMEM((NC1, NB1), jnp.int32),
            pltpu.VMEM((NC1, NB1), jnp.int32),
            pltpu.VMEM((NB1, 16), jnp.float32),
            pltpu.VMEM((NB1, 16), jnp.float32),
            pltpu.VMEM((NB1, 16), jnp.float32),
            pltpu.VMEM((NB1, 16), jnp.float32),
            pltpu.VMEM((NB1, 16), jnp.float32),
            pltpu.VMEM((16,), jnp.float32),
            pltpu.VMEM_SHARED((NPAD, 16), jnp.float32),
            pltpu.SemaphoreType.DMA,
            pltpu.SemaphoreType.DMA,
            pltpu.SemaphoreType.DMA,
            pltpu.SemaphoreType.DMA,
        ],
    )(src2, dst2, S, D, amax16, zd)


def _recip_body(d_ref, r_ref):
    r_ref[...] = 1.0 / (d_ref[0] + d_ref[1] + 1e-16)


def _recip(denom):
    return pl.pallas_call(
        _recip_body,
        out_shape=jax.ShapeDtypeStruct((NPAD, 16), jnp.float32),
    )(denom)


def _sc_msgs_body(src_hbm, dst2_hbm, r_hbm, x_hbm, exf_hbm, zo_hbm,
                  opart_hbm,
                  dall, si0, si1, pr0, pr1, xb0, xb1, ev0, ev1, cb0, cb1, osh,
                  semS0, semS1, semR0, semR1, semX0, semX1, semE0, semE1,
                  semC0, semC1, semY0, semY1):
    c = lax.axis_index("c")
    s = lax.axis_index("s")
    w = s * 2 + c
    rows = NPAD // 16
    r0 = s * rows
    pltpu.sync_copy(zo_hbm.at[pl.ds(r0, rows)], osh.at[pl.ds(r0, rows)])
    pltpu.sync_copy(dst2_hbm.at[pl.ds(w * NC2, NC2)], dall)
    plsc.subcore_barrier()
    sis = (si0, si1)
    prs = (pr0, pr1)
    xbs = (xb0, xb1)
    evs = (ev0, ev1)
    cbs = (cb0, cb1)
    semCs = (semC0, semC1)
    semYs = (semY0, semY1)
    H2 = NB2 // 2
    semSs = (semS0, semS1)
    semRs = (semR0, semR1)
    semXs = (semX0, semX1)
    semEs = (semE0, semE1)
    hsel = [jnp.full((16,), h, dtype=jnp.int32) for h in range(HEADS)]
    base_e = w * NC2 * NB2

    def fire_si(j, slot):
        pltpu.async_copy(src_hbm.at[pl.ds(base_e + j * NB2, NB2)],
                         sis[slot], semSs[slot])

    def wait_si(j, slot):
        pltpu.make_async_copy(src_hbm.at[pl.ds(base_e + j * NB2, NB2)],
                              sis[slot], semSs[slot]).wait()

    def fire_g(j, b, slot):
        gb = base_e + j * NB2
        pltpu.async_copy(r_hbm.at[dall.at[j]], prs[b], semRs[b])
        pltpu.async_copy(x_hbm.at[sis[slot].at[pl.ds(0, H2)]],
                         xbs[b].at[pl.ds(0, H2)], semXs[b])
        pltpu.async_copy(x_hbm.at[sis[slot].at[pl.ds(H2, H2)]],
                         xbs[b].at[pl.ds(H2, H2)], semYs[b])
        pltpu.async_copy(exf_hbm.at[pl.ds(gb, NB2)], evs[b], semEs[b])

    def wait_g(j, b):
        gb = base_e + j * NB2
        pltpu.make_async_copy(r_hbm.at[dall.at[j]], prs[b], semRs[b]).wait()
        pltpu.make_async_copy(x_hbm.at[sis[0].at[pl.ds(0, H2)]],
                              xbs[b].at[pl.ds(0, H2)], semXs[b]).wait()
        pltpu.make_async_copy(x_hbm.at[sis[0].at[pl.ds(H2, H2)]],
                              xbs[b].at[pl.ds(H2, H2)], semYs[b]).wait()
        pltpu.make_async_copy(exf_hbm.at[pl.ds(gb, NB2)], evs[b],
                              semEs[b]).wait()

    fire_si(0, 0)
    wait_si(0, 0)
    fire_si(1, 1)
    fire_g(0, 0, 0)

    def outer(i2, carry):
        for b in range(2):
            j = i2 * 2 + b
            nxt = j + 1

            @pl.when(nxt < NC2)
            def _():
                wait_si(nxt, 1 - b)

            wait_g(j, b)

            @pl.when(nxt < NC2)
            def _():
                fire_g(nxt, 1 - b, 1 - b)

            @pl.when(j + 2 < NC2)
            def _():
                fire_si(j + 2, b)

            pr = prs[b]
            xb = xbs[b]
            ev = evs[b]
            cb = cbs[b]

            @pl.when(j >= 2)
            def _():
                pltpu.make_async_copy(cb, osh.at[dall.at[j - 2]],
                                      semCs[b]).wait()

            def e_body(e, cc):
                att = ev[e] * pr[e]
                for h in range(HEADS):
                    ab = att.at[hsel[h]].get(mode="promise_in_bounds")
                    base = h * DIM
                    for g in range(4):
                        x32 = xb[e, pl.ds(base + g * 32, 32)]
                        lo, hi = plsc.unpack(
                            x32, format=plsc.PackFormat.INTERLEAVED)
                        if h == 0:
                            cb[e, pl.ds(g * 32, 16)] = ab * lo
                            cb[e, pl.ds(g * 32 + 16, 16)] = ab * hi
                        else:
                            cb[e, pl.ds(g * 32, 16)] += ab * lo
                            cb[e, pl.ds(g * 32 + 16, 16)] += ab * hi
                return cc

            lax.fori_loop(0, NB2, e_body, 0)

            pltpu.async_copy(cb, osh.at[dall.at[j]], semCs[b], add=True)
        return carry

    lax.fori_loop(0, NC2 // 2, outer, 0)
    pltpu.make_async_copy(cbs[0], osh.at[dall.at[NC2 - 2]], semCs[0]).wait()
    pltpu.make_async_copy(cbs[1], osh.at[dall.at[NC2 - 1]], semCs[1]).wait()
    plsc.subcore_barrier()
    pltpu.sync_copy(osh.at[pl.ds(r0, rows)], opart_hbm.at[c, pl.ds(r0, rows)])


def _sc_msgs(src, dst2, R, xp, exf, zo):
    mesh = plsc.VectorSubcoreMesh(core_axis_name="c", subcore_axis_name="s")
    return pl.kernel(
        _sc_msgs_body,
        out_type=jax.ShapeDtypeStruct((2, NPAD, DIM), jnp.float32),
        mesh=mesh,
        compiler_params=_SC_PARAMS2,
        scratch_types=[
            pltpu.VMEM((NC2, NB2), jnp.int32),
            pltpu.VMEM((NB2,), jnp.int32),
            pltpu.VMEM((NB2,), jnp.int32),
            pltpu.VMEM((NB2, 16), jnp.float32),
            pltpu.VMEM((NB2, 16), jnp.float32),
            pltpu.VMEM((NB2, HEADS * DIM), jnp.bfloat16),
            pltpu.VMEM((NB2, HEADS * DIM), jnp.bfloat16),
            pltpu.VMEM((NB2, 16), jnp.float32),
            pltpu.VMEM((NB2, 16), jnp.float32),
            pltpu.VMEM((NB2, DIM), jnp.float32),
            pltpu.VMEM((NB2, DIM), jnp.float32),
            pltpu.VMEM_SHARED((NPAD, DIM), jnp.float32),
            pltpu.SemaphoreType.DMA,
            pltpu.SemaphoreType.DMA,
            pltpu.SemaphoreType.DMA,
            pltpu.SemaphoreType.DMA,
            pltpu.SemaphoreType.DMA,
            pltpu.SemaphoreType.DMA,
            pltpu.SemaphoreType.DMA,
            pltpu.SemaphoreType.DMA,
            pltpu.SemaphoreType.DMA,
            pltpu.SemaphoreType.DMA,
            pltpu.SemaphoreType.DMA,
            pltpu.SemaphoreType.DMA,
        ],
    )(src, dst2, R, xp, exf, zo)


def _final_body(op_ref, bias_ref, o_ref):
    o_ref[...] = (op_ref[0] + op_ref[1]) * (1.0 / HEADS) + bias_ref[...]


def _final(opart, bias):
    nblk = 10
    B = N_NODES // nblk
    return pl.pallas_call(
        _final_body,
        grid=(nblk,),
        in_specs=[
            pl.BlockSpec((2, B, DIM), lambda i: (0, i, 0)),
            pl.BlockSpec((1, DIM), lambda i: (0, 0)),
        ],
        out_specs=pl.BlockSpec((B, DIM), lambda i: (i, 0)),
        out_shape=jax.ShapeDtypeStruct((N_NODES, DIM), jnp.float32),
    )(opart, bias.reshape(1, DIM))


def kernel(x, edge_index, W, att_src, att_dst, bias):
    N = N_NODES
    E2 = edge_index.shape[1] + N

    loop = jnp.arange(N, dtype=jnp.int32)
    src = jnp.concatenate([edge_index[0].astype(jnp.int32), loop,
                           jnp.zeros((EP - E2,), jnp.int32)])
    dst = jnp.concatenate([edge_index[1].astype(jnp.int32), loop,
                           jnp.full((EP - E2,), N, jnp.int32)])
    src1 = src.reshape(EP // NB1, NB1)
    dst1 = dst.reshape(EP // NB1, NB1)
    dst2 = dst.reshape(EP // NB2, NB2)

    att_cat = jnp.concatenate([att_src, att_dst], axis=0)
    xp, acat, amaxo = _project(x, W, att_cat)

    pad = ((0, NPAD - N), (0, 0))
    S = jnp.pad(jnp.concatenate([acat[:, :8], acat[:, :8]], axis=1), pad)
    Dd = jnp.pad(jnp.concatenate([acat[:, 8:], acat[:, 8:]], axis=1), pad)
    amax16 = jnp.concatenate([amaxo[0, :8], amaxo[0, :8]])

    zd = jnp.zeros((NPAD, 16), jnp.float32)
    denom, exf = _sc_denom(src1, dst1, S, Dd, amax16, zd)
    R = _recip(denom)

    xsw = (xp.reshape(N, HEADS, 4, 2, 16).transpose(0, 1, 2, 4, 3)
           .reshape(N, HEADS * DIM).astype(jnp.bfloat16))

    zo = jnp.zeros((NPAD, DIM), jnp.float32)
    opart = _sc_msgs(src, dst2, R, xsw, exf, zo)
    return _final(opart, bias)


# speedup vs baseline: 1.1155x; 1.1155x over previous
import jax
import jax.numpy as jnp
from jax import lax
from jax.experimental import pallas as pl
from jax.experimental.pallas import tpu as pltpu
from jax.experimental.pallas import tpu_sc as plsc

N_NODES = 10000
DIM = 128
HEADS = 8
NPAD = 10112
NWORK = 32
NB1 = 96
NB2 = 16
EP = 331776
NC1 = EP // (NWORK * NB1)
NC2 = EP // (NWORK * NB2)

_SC_PARAMS = pltpu.CompilerParams(use_tc_tiling_on_sc=False)
_SC_PARAMS2 = pltpu.CompilerParams(use_tc_tiling_on_sc=False,
                                   needs_layout_passes=False)


def _leaky(v):
    return jnp.where(v > 0, v, v * 0.2)


def _proj_body(x_ref, w_ref, attcat_ref, xp_ref, acat_ref, amax_ref):
    i = pl.program_id(0)
    x_blk = x_ref[...]
    xp_ref[...] = lax.dot_general(x_blk, w_ref[...], (((1,), (1,)), ((), ())),
                                  preferred_element_type=jnp.float32)
    rows = []
    for k in range(16):
        wk = w_ref[pl.ds((k % HEADS) * DIM, DIM), :]
        rows.append(jnp.dot(attcat_ref[k:k + 1, :], wk,
                            preferred_element_type=jnp.float32))
    vcat = jnp.concatenate(rows, axis=0)
    acat = lax.dot_general(x_blk, vcat, (((1,), (1,)), ((), ())),
                           preferred_element_type=jnp.float32)
    acat_ref[...] = acat
    cur = jnp.max(acat, axis=0, keepdims=True)

    @pl.when(i == 0)
    def _():
        amax_ref[...] = cur

    @pl.when(i > 0)
    def _():
        amax_ref[...] = jnp.maximum(amax_ref[...], cur)


def _project(x, W, att_cat):
    nblk = 10
    B = N_NODES // nblk
    return pl.pallas_call(
        _proj_body,
        grid=(nblk,),
        in_specs=[
            pl.BlockSpec((B, DIM), lambda i: (i, 0)),
            pl.BlockSpec((HEADS * DIM, DIM), lambda i: (0, 0)),
            pl.BlockSpec((16, DIM), lambda i: (0, 0)),
        ],
        out_specs=[
            pl.BlockSpec((B, HEADS * DIM), lambda i: (i, 0)),
            pl.BlockSpec((B, 16), lambda i: (i, 0)),
            pl.BlockSpec((1, 16), lambda i: (0, 0)),
        ],
        out_shape=[
            jax.ShapeDtypeStruct((N_NODES, HEADS * DIM), jnp.float32),
            jax.ShapeDtypeStruct((N_NODES, 16), jnp.float32),
            jax.ShapeDtypeStruct((1, 16), jnp.float32),
        ],
    )(x, W, att_cat)


def _sc_denom_body(src2_hbm, dst2_hbm, s_hbm, d_hbm, amax_hbm, zd_hbm,
                   denom_hbm, exf_hbm,
                   sall, dall, pa0, pa1, pb0, pb1, exb, amv, dsh,
                   semA0, semA1, semB0, semB1):
    c = lax.axis_index("c")
    s = lax.axis_index("s")
    w = s * 2 + c
    rows = NPAD // 16
    r0 = s * rows
    pltpu.sync_copy(zd_hbm.at[pl.ds(r0, rows)], dsh.at[pl.ds(r0, rows)])
    pltpu.sync_copy(amax_hbm, amv)
    pltpu.sync_copy(src2_hbm.at[pl.ds(w * NC1, NC1)], sall)
    pltpu.sync_copy(dst2_hbm.at[pl.ds(w * NC1, NC1)], dall)
    plsc.subcore_barrier()
    av = amv[...]
    pas = (pa0, pa1)
    pbs = (pb0, pb1)
    semAs = (semA0, semA1)
    semBs = (semB0, semB1)

    def fire(j, b):
        pltpu.async_copy(s_hbm.at[sall.at[j]], pas[b], semAs[b])
        pltpu.async_copy(d_hbm.at[dall.at[j]], pbs[b], semBs[b])

    fire(0, 0)

    def outer(i2, carry):
        for b in range(2):
            j = i2 * 2 + b
            pltpu.make_async_copy(s_hbm.at[sall.at[j]], pas[b], semAs[b]).wait()
            pltpu.make_async_copy(d_hbm.at[dall.at[j]], pbs[b], semBs[b]).wait()
            nxt = j + 1

            @pl.when(nxt < NC1)
            def _():
                fire(nxt, 1 - b)

            pa = pas[b]
            pb = pbs[b]

            @plsc.parallel_loop(0, NB1, 1, unroll=4)
            def _(e):
                va = pa[e]
                vb = pb[e]
                al = _leaky(va + vb)
                bd = _leaky(av + vb)
                exb[e] = jnp.exp(al - bd)
            pltpu.sync_copy(exb, dsh.at[dall.at[j]], add=True)
            gb = (w * NC1 + j) * NB1
            pltpu.sync_copy(exb, exf_hbm.at[pl.ds(gb, NB1)])
        return carry

    lax.fori_loop(0, NC1 // 2, outer, 0)
    plsc.subcore_barrier()
    pltpu.sync_copy(dsh.at[pl.ds(r0, rows)], denom_hbm.at[c, pl.ds(r0, rows)])


def _sc_denom(src2, dst2, S, D, amax16, zd):
    mesh = plsc.VectorSubcoreMesh(core_axis_name="c", subcore_axis_name="s")
    return pl.kernel(
        _sc_denom_body,
        out_type=[
            jax.ShapeDtypeStruct((2, NPAD, 16), jnp.float32),
            jax.ShapeDtypeStruct((EP, 16), jnp.float32),
        ],
        mesh=mesh,
        compiler_params=_SC_PARAMS,
        scratch_types=[
            pltpu.VMEM((NC1, NB1), jnp.int32),
            pltpu.VMEM((NC1, NB1), jnp.int32),
            pltpu.VMEM((NB1, 16), jnp.float32),
            pltpu.VMEM((NB1, 16), jnp.float32),
            pltpu.VMEM((NB1, 16), jnp.float32),
            pltpu.VMEM((NB1, 16), jnp.float32),
            pltpu.VMEM((NB1, 16), jnp.float32),
            pltpu.VMEM((16,), jnp.float32),
            pltpu.VMEM_SHARED((NPAD, 16), jnp.float32),
            pltpu.SemaphoreType.DMA,
            pltpu.SemaphoreType.DMA,
            pltpu.SemaphoreType.DMA,
            pltpu.SemaphoreType.DMA,
        ],
    )(src2, dst2, S, D, amax16, zd)


def _recip_body(d_ref, r_ref):
    r_ref[...] = 1.0 / (d_ref[0] + d_ref[1] + 1e-16)


def _recip(denom):
    return pl.pallas_call(
        _recip_body,
        out_shape=jax.ShapeDtypeStruct((NPAD, 16), jnp.float32),
    )(denom)


def _sc_msgs_body(src_hbm, dst2_hbm, r_hbm, x_hbm, exf_hbm, zo_hbm,
                  opart_hbm,
                  dall, si0, si1, pr0, pr1, xb0, xb1, ev0, ev1, cb0, cb1, osh,
                  semS0, semS1, semR0, semR1, semX0, semX1, semE0, semE1,
                  semC0, semC1, semY0, semY1):
    c = lax.axis_index("c")
    s = lax.axis_index("s")
    w = s * 2 + c
    rows = NPAD // 16
    r0 = s * rows
    pltpu.sync_copy(zo_hbm.at[pl.ds(r0, rows)], osh.at[pl.ds(r0, rows)])
    pltpu.sync_copy(dst2_hbm.at[pl.ds(w * NC2, NC2)], dall)
    plsc.subcore_barrier()
    sis = (si0, si1)
    prs = (pr0, pr1)
    xbs = (xb0, xb1)
    evs = (ev0, ev1)
    cbs = (cb0, cb1)
    semCs = (semC0, semC1)
    semYs = (semY0, semY1)
    H2 = NB2 // 2
    semSs = (semS0, semS1)
    semRs = (semR0, semR1)
    semXs = (semX0, semX1)
    semEs = (semE0, semE1)
    hsel = [jnp.full((16,), h, dtype=jnp.int32) for h in range(HEADS)]
    base_e = w * NC2 * NB2

    def fire_si(j, slot):
        pltpu.async_copy(src_hbm.at[pl.ds(base_e + j * NB2, NB2)],
                         sis[slot], semSs[slot])

    def wait_si(j, slot):
        pltpu.make_async_copy(src_hbm.at[pl.ds(base_e + j * NB2, NB2)],
                              sis[slot], semSs[slot]).wait()

    def fire_g(j, b, slot):
        gb = base_e + j * NB2
        pltpu.async_copy(r_hbm.at[dall.at[j]], prs[b], semRs[b])
        pltpu.async_copy(x_hbm.at[sis[slot].at[pl.ds(0, H2)]],
                         xbs[b].at[pl.ds(0, H2)], semXs[b])
        pltpu.async_copy(x_hbm.at[sis[slot].at[pl.ds(H2, H2)]],
                         xbs[b].at[pl.ds(H2, H2)], semYs[b])
        pltpu.async_copy(exf_hbm.at[pl.ds(gb, NB2)], evs[b], semEs[b])

    def wait_g(j, b):
        gb = base_e + j * NB2
        pltpu.make_async_copy(r_hbm.at[dall.at[j]], prs[b], semRs[b]).wait()
        pltpu.make_async_copy(x_hbm.at[sis[0].at[pl.ds(0, H2)]],
                              xbs[b].at[pl.ds(0, H2)], semXs[b]).wait()
        pltpu.make_async_copy(x_hbm.at[sis[0].at[pl.ds(H2, H2)]],
                              xbs[b].at[pl.ds(H2, H2)], semYs[b]).wait()
        pltpu.make_async_copy(exf_hbm.at[pl.ds(gb, NB2)], evs[b],
                              semEs[b]).wait()

    fire_si(0, 0)
    wait_si(0, 0)
    fire_si(1, 1)
    fire_g(0, 0, 0)

    def outer(i2, carry):
        for b in range(2):
            j = i2 * 2 + b
            nxt = j + 1

            @pl.when(nxt < NC2)
            def _():
                wait_si(nxt, 1 - b)

            wait_g(j, b)

            @pl.when(nxt < NC2)
            def _():
                fire_g(nxt, 1 - b, 1 - b)

            @pl.when(j + 2 < NC2)
            def _():
                fire_si(j + 2, b)

            pr = prs[b]
            xb = xbs[b]
            ev = evs[b]
            cb = cbs[b]

            @pl.when(j >= 2)
            def _():
                pltpu.make_async_copy(cb, osh.at[dall.at[j - 2]],
                                      semCs[b]).wait()

            def e_body(e, cc):
                att = ev[e] * pr[e]
                for h in range(HEADS):
                    ab = att.at[hsel[h]].get(mode="promise_in_bounds")
                    base = h * DIM
                    for g in range(4):
                        x32 = xb[e, pl.ds(base + g * 32, 32)]
                        lo, hi = plsc.unpack(
                            x32, format=plsc.PackFormat.INTERLEAVED)
                        if h == 0:
                            cb[e, pl.ds(g * 32, 16)] = ab * lo
                            cb[e, pl.ds(g * 32 + 16, 16)] = ab * hi
                        else:
                            cb[e, pl.ds(g * 32, 16)] += ab * lo
                            cb[e, pl.ds(g * 32 + 16, 16)] += ab * hi
                return cc

            lax.fori_loop(0, NB2, e_body, 0)

            pltpu.async_copy(cb, osh.at[dall.at[j]], semCs[b], add=True)
        return carry

    lax.fori_loop(0, NC2 // 2, outer, 0)
    pltpu.make_async_copy(cbs[0], osh.at[dall.at[NC2 - 2]], semCs[0]).wait()
    pltpu.make_async_copy(cbs[1], osh.at[dall.at[NC2 - 1]], semCs[1]).wait()
    plsc.subcore_barrier()
    pltpu.sync_copy(osh.at[pl.ds(r0, rows)], opart_hbm.at[c, pl.ds(r0, rows)])


def _sc_msgs(src, dst2, R, xp, exf, zo):
    mesh = plsc.VectorSubcoreMesh(core_axis_name="c", subcore_axis_name="s")
    return pl.kernel(
        _sc_msgs_body,
        out_type=jax.ShapeDtypeStruct((2, NPAD, DIM), jnp.float32),
        mesh=mesh,
        compiler_params=_SC_PARAMS2,
        scratch_types=[
            pltpu.VMEM((NC2, NB2), jnp.int32),
            pltpu.VMEM((NB2,), jnp.int32),
            pltpu.VMEM((NB2,), jnp.int32),
            pltpu.VMEM((NB2, 16), jnp.float32),
            pltpu.VMEM((NB2, 16), jnp.float32),
            pltpu.VMEM((NB2, HEADS * DIM), jnp.bfloat16),
            pltpu.VMEM((NB2, HEADS * DIM), jnp.bfloat16),
            pltpu.VMEM((NB2, 16), jnp.float32),
            pltpu.VMEM((NB2, 16), jnp.float32),
            pltpu.VMEM((NB2, DIM), jnp.float32),
            pltpu.VMEM((NB2, DIM), jnp.float32),
            pltpu.VMEM_SHARED((NPAD, DIM), jnp.float32),
            pltpu.SemaphoreType.DMA,
            pltpu.SemaphoreType.DMA,
            pltpu.SemaphoreType.DMA,
            pltpu.SemaphoreType.DMA,
            pltpu.SemaphoreType.DMA,
            pltpu.SemaphoreType.DMA,
            pltpu.SemaphoreType.DMA,
            pltpu.SemaphoreType.DMA,
            pltpu.SemaphoreType.DMA,
            pltpu.SemaphoreType.DMA,
            pltpu.SemaphoreType.DMA,
            pltpu.SemaphoreType.DMA,
        ],
    )(src, dst2, R, xp, exf, zo)


def _final_body(op_ref, bias_ref, o_ref):
    o_ref[...] = (op_ref[0] + op_ref[1]) * (1.0 / HEADS) + bias_ref[...]


def _final(opart, bias):
    nblk = 10
    B = N_NODES // nblk
    return pl.pallas_call(
        _final_body,
        grid=(nblk,),
        in_specs=[
            pl.BlockSpec((2, B, DIM), lambda i: (0, i, 0)),
            pl.BlockSpec((1, DIM), lambda i: (0, 0)),
        ],
        out_specs=pl.BlockSpec((B, DIM), lambda i: (i, 0)),
        out_shape=jax.ShapeDtypeStruct((N_NODES, DIM), jnp.float32),
    )(opart, bias.reshape(1, DIM))


def kernel(x, edge_index, W, att_src, att_dst, bias):
    N = N_NODES
    E2 = edge_index.shape[1] + N

    loop = jnp.arange(N, dtype=jnp.int32)
    src = jnp.concatenate([edge_index[0].astype(jnp.int32), loop,
                           jnp.zeros((EP - E2,), jnp.int32)])
    dst = jnp.concatenate([edge_index[1].astype(jnp.int32), loop,
                           jnp.full((EP - E2,), N, jnp.int32)])
    src1 = src.reshape(EP // NB1, NB1)
    dst1 = dst.reshape(EP // NB1, NB1)
    dst2 = dst.reshape(EP // NB2, NB2)

    att_cat = jnp.concatenate([att_src, att_dst], axis=0)
    xp, acat, amaxo = _project(x, W, att_cat)

    pad = ((0, NPAD - N), (0, 0))
    S = jnp.pad(jnp.concatenate([acat[:, :8], acat[:, :8]], axis=1), pad)
    Dd = jnp.pad(jnp.concatenate([acat[:, 8:], acat[:, 8:]], axis=1), pad)
    amax16 = jnp.concatenate([amaxo[0, :8], amaxo[0, :8]])

    zd = jnp.zeros((NPAD, 16), jnp.float32)
    denom, exf = _sc_denom(src1, dst1, S, Dd, amax16, zd)
    R = _recip(denom)

    xsw = (xp.reshape(N, HEADS, 4, 2, 16).transpose(0, 1, 2, 4, 3)
           .reshape(N, HEADS * DIM).astype(jnp.bfloat16))

    zo = jnp.zeros((NPAD, DIM), jnp.float32)
    opart = _sc_msgs(src, dst2, R, xsw, exf, zo)
    return _final(opart, bias)


# speedup vs baseline: 1.1187x; 1.0029x over previous
import jax
import jax.numpy as jnp
from jax import lax
from jax.experimental import pallas as pl
from jax.experimental.pallas import tpu as pltpu
from jax.experimental.pallas import tpu_sc as plsc

N_NODES = 10000
DIM = 128
HEADS = 8
NPAD = 10112
NWORK = 32
NB1 = 96
NB2 = 24
EP = 331776
NC1 = EP // (NWORK * NB1)
NC2 = EP // (NWORK * NB2)

_SC_PARAMS = pltpu.CompilerParams(use_tc_tiling_on_sc=False)
_SC_PARAMS2 = pltpu.CompilerParams(use_tc_tiling_on_sc=False,
                                   needs_layout_passes=False)


def _leaky(v):
    return jnp.where(v > 0, v, v * 0.2)


def _proj_body(x_ref, w_ref, attcat_ref, xp_ref, acat_ref, amax_ref):
    i = pl.program_id(0)
    x_blk = x_ref[...]
    xp_ref[...] = lax.dot_general(x_blk, w_ref[...], (((1,), (1,)), ((), ())),
                                  preferred_element_type=jnp.float32)
    rows = []
    for k in range(16):
        wk = w_ref[pl.ds((k % HEADS) * DIM, DIM), :]
        rows.append(jnp.dot(attcat_ref[k:k + 1, :], wk,
                            preferred_element_type=jnp.float32))
    vcat = jnp.concatenate(rows, axis=0)
    acat = lax.dot_general(x_blk, vcat, (((1,), (1,)), ((), ())),
                           preferred_element_type=jnp.float32)
    acat_ref[...] = acat
    cur = jnp.max(acat, axis=0, keepdims=True)

    @pl.when(i == 0)
    def _():
        amax_ref[...] = cur

    @pl.when(i > 0)
    def _():
        amax_ref[...] = jnp.maximum(amax_ref[...], cur)


def _project(x, W, att_cat):
    nblk = 10
    B = N_NODES // nblk
    return pl.pallas_call(
        _proj_body,
        grid=(nblk,),
        in_specs=[
            pl.BlockSpec((B, DIM), lambda i: (i, 0)),
            pl.BlockSpec((HEADS * DIM, DIM), lambda i: (0, 0)),
            pl.BlockSpec((16, DIM), lambda i: (0, 0)),
        ],
        out_specs=[
            pl.BlockSpec((B, HEADS * DIM), lambda i: (i, 0)),
            pl.BlockSpec((B, 16), lambda i: (i, 0)),
            pl.BlockSpec((1, 16), lambda i: (0, 0)),
        ],
        out_shape=[
            jax.ShapeDtypeStruct((N_NODES, HEADS * DIM), jnp.float32),
            jax.ShapeDtypeStruct((N_NODES, 16), jnp.float32),
            jax.ShapeDtypeStruct((1, 16), jnp.float32),
        ],
    )(x, W, att_cat)


def _sc_denom_body(src2_hbm, dst2_hbm, s_hbm, d_hbm, amax_hbm, zd_hbm,
                   denom_hbm, exf_hbm,
                   sall, dall, pa0, pa1, pb0, pb1, exb, amv, dsh,
                   semA0, semA1, semB0, semB1):
    c = lax.axis_index("c")
    s = lax.axis_index("s")
    w = s * 2 + c
    rows = NPAD // 16
    r0 = s * rows
    pltpu.sync_copy(zd_hbm.at[pl.ds(r0, rows)], dsh.at[pl.ds(r0, rows)])
    pltpu.sync_copy(amax_hbm, amv)
    pltpu.sync_copy(src2_hbm.at[pl.ds(w * NC1, NC1)], sall)
    pltpu.sync_copy(dst2_hbm.at[pl.ds(w * NC1, NC1)], dall)
    plsc.subcore_barrier()
    av = amv[...]
    pas = (pa0, pa1)
    pbs = (pb0, pb1)
    semAs = (semA0, semA1)
    semBs = (semB0, semB1)

    def fire(j, b):
        pltpu.async_copy(s_hbm.at[sall.at[j]], pas[b], semAs[b])
        pltpu.async_copy(d_hbm.at[dall.at[j]], pbs[b], semBs[b])

    fire(0, 0)

    def outer(i2, carry):
        for b in range(2):
            j = i2 * 2 + b
            pltpu.make_async_copy(s_hbm.at[sall.at[j]], pas[b], semAs[b]).wait()
            pltpu.make_async_copy(d_hbm.at[dall.at[j]], pbs[b], semBs[b]).wait()
            nxt = j + 1

            @pl.when(nxt < NC1)
            def _():
                fire(nxt, 1 - b)

            pa = pas[b]
            pb = pbs[b]

            @plsc.parallel_loop(0, NB1, 1, unroll=4)
            def _(e):
                va = pa[e]
                vb = pb[e]
                al = _leaky(va + vb)
                bd = _leaky(av + vb)
                exb[e] = jnp.exp(al - bd)
            pltpu.sync_copy(exb, dsh.at[dall.at[j]], add=True)
            gb = (w * NC1 + j) * NB1
            pltpu.sync_copy(exb, exf_hbm.at[pl.ds(gb, NB1)])
        return carry

    lax.fori_loop(0, NC1 // 2, outer, 0)
    plsc.subcore_barrier()
    pltpu.sync_copy(dsh.at[pl.ds(r0, rows)], denom_hbm.at[c, pl.ds(r0, rows)])


def _sc_denom(src2, dst2, S, D, amax16, zd):
    mesh = plsc.VectorSubcoreMesh(core_axis_name="c", subcore_axis_name="s")
    return pl.kernel(
        _sc_denom_body,
        out_type=[
            jax.ShapeDtypeStruct((2, NPAD, 16), jnp.float32),
            jax.ShapeDtypeStruct((EP, 16), jnp.float32),
        ],
        mesh=mesh,
        compiler_params=_SC_PARAMS,
        scratch_types=[
            pltpu.VMEM((NC1, NB1), jnp.int32),
            pltpu.VMEM((NC1, NB1), jnp.int32),
            pltpu.VMEM((NB1, 16), jnp.float32),
            pltpu.VMEM((NB1, 16), jnp.float32),
            pltpu.VMEM((NB1, 16), jnp.float32),
            pltpu.VMEM((NB1, 16), jnp.float32),
            pltpu.VMEM((NB1, 16), jnp.float32),
            pltpu.VMEM((16,), jnp.float32),
            pltpu.VMEM_SHARED((NPAD, 16), jnp.float32),
            pltpu.SemaphoreType.DMA,
            pltpu.SemaphoreType.DMA,
            pltpu.SemaphoreType.DMA,
            pltpu.SemaphoreType.DMA,
        ],
    )(src2, dst2, S, D, amax16, zd)


def _recip_body(d_ref, r_ref):
    r_ref[...] = 1.0 / (d_ref[0] + d_ref[1] + 1e-16)


def _recip(denom):
    return pl.pallas_call(
        _recip_body,
        out_shape=jax.ShapeDtypeStruct((NPAD, 16), jnp.float32),
    )(denom)


def _sc_msgs_body(src_hbm, dst2_hbm, r_hbm, x_hbm, exf_hbm, zo_hbm,
                  opart_hbm,
                  dall, si0, si1, pr0, pr1, xb0, xb1, ev0, ev1, cb0, cb1, osh,
                  semS0, semS1, semR0, semR1, semX0, semX1, semE0, semE1,
                  semC0, semC1, semY0, semY1):
    c = lax.axis_index("c")
    s = lax.axis_index("s")
    w = s * 2 + c
    rows = NPAD // 16
    r0 = s * rows
    pltpu.sync_copy(zo_hbm.at[pl.ds(r0, rows)], osh.at[pl.ds(r0, rows)])
    pltpu.sync_copy(dst2_hbm.at[pl.ds(w * NC2, NC2)], dall)
    plsc.subcore_barrier()
    sis = (si0, si1)
    prs = (pr0, pr1)
    xbs = (xb0, xb1)
    evs = (ev0, ev1)
    cbs = (cb0, cb1)
    semCs = (semC0, semC1)
    semYs = (semY0, semY1)
    H2 = NB2 // 2
    semSs = (semS0, semS1)
    semRs = (semR0, semR1)
    semXs = (semX0, semX1)
    semEs = (semE0, semE1)
    hsel = [jnp.full((16,), h, dtype=jnp.int32) for h in range(HEADS)]
    base_e = w * NC2 * NB2

    def fire_si(j, slot):
        pltpu.async_copy(src_hbm.at[pl.ds(base_e + j * NB2, NB2)],
                         sis[slot], semSs[slot])

    def wait_si(j, slot):
        pltpu.make_async_copy(src_hbm.at[pl.ds(base_e + j * NB2, NB2)],
                              sis[slot], semSs[slot]).wait()

    def fire_g(j, b, slot):
        gb = base_e + j * NB2
        pltpu.async_copy(r_hbm.at[dall.at[j]], prs[b], semRs[b])
        pltpu.async_copy(x_hbm.at[sis[slot]], xbs[b], semXs[b])
        pltpu.async_copy(exf_hbm.at[pl.ds(gb, NB2)], evs[b], semEs[b])

    def wait_g(j, b):
        gb = base_e + j * NB2
        pltpu.make_async_copy(r_hbm.at[dall.at[j]], prs[b], semRs[b]).wait()
        pltpu.make_async_copy(x_hbm.at[sis[0]], xbs[b], semXs[b]).wait()
        pltpu.make_async_copy(exf_hbm.at[pl.ds(gb, NB2)], evs[b],
                              semEs[b]).wait()

    fire_si(0, 0)
    wait_si(0, 0)
    fire_si(1, 1)
    fire_g(0, 0, 0)

    def outer(i2, carry):
        for b in range(2):
            j = i2 * 2 + b
            nxt = j + 1

            @pl.when(nxt < NC2)
            def _():
                wait_si(nxt, 1 - b)

            wait_g(j, b)

            @pl.when(nxt < NC2)
            def _():
                fire_g(nxt, 1 - b, 1 - b)

            @pl.when(j + 2 < NC2)
            def _():
                fire_si(j + 2, b)

            pr = prs[b]
            xb = xbs[b]
            ev = evs[b]
            cb = cbs[b]

            @pl.when(j >= 2)
            def _():
                pltpu.make_async_copy(cb, osh.at[dall.at[j - 2]],
                                      semCs[b]).wait()

            def e_body(e, cc):
                att = ev[e] * pr[e]
                for h in range(HEADS):
                    ab = att.at[hsel[h]].get(mode="promise_in_bounds")
                    base = h * DIM
                    for g in range(4):
                        x32 = xb[e, pl.ds(base + g * 32, 32)]
                        lo, hi = plsc.unpack(
                            x32, format=plsc.PackFormat.INTERLEAVED)
                        if h == 0:
                            cb[e, pl.ds(g * 32, 16)] = ab * lo
                            cb[e, pl.ds(g * 32 + 16, 16)] = ab * hi
                        else:
                            cb[e, pl.ds(g * 32, 16)] += ab * lo
                            cb[e, pl.ds(g * 32 + 16, 16)] += ab * hi
                return cc

            lax.fori_loop(0, NB2, e_body, 0)

            pltpu.async_copy(cb, osh.at[dall.at[j]], semCs[b], add=True)
        return carry

    lax.fori_loop(0, NC2 // 2, outer, 0)
    pltpu.make_async_copy(cbs[0], osh.at[dall.at[NC2 - 2]], semCs[0]).wait()
    pltpu.make_async_copy(cbs[1], osh.at[dall.at[NC2 - 1]], semCs[1]).wait()
    plsc.subcore_barrier()
    pltpu.sync_copy(osh.at[pl.ds(r0, rows)], opart_hbm.at[c, pl.ds(r0, rows)])


def _sc_msgs(src, dst2, R, xp, exf, zo):
    mesh = plsc.VectorSubcoreMesh(core_axis_name="c", subcore_axis_name="s")
    return pl.kernel(
        _sc_msgs_body,
        out_type=jax.ShapeDtypeStruct((2, NPAD, DIM), jnp.float32),
        mesh=mesh,
        compiler_params=_SC_PARAMS2,
        scratch_types=[
            pltpu.VMEM((NC2, NB2), jnp.int32),
            pltpu.VMEM((NB2,), jnp.int32),
            pltpu.VMEM((NB2,), jnp.int32),
            pltpu.VMEM((NB2, 16), jnp.float32),
            pltpu.VMEM((NB2, 16), jnp.float32),
            pltpu.VMEM((NB2, HEADS * DIM), jnp.bfloat16),
            pltpu.VMEM((NB2, HEADS * DIM), jnp.bfloat16),
            pltpu.VMEM((NB2, 16), jnp.float32),
            pltpu.VMEM((NB2, 16), jnp.float32),
            pltpu.VMEM((NB2, DIM), jnp.float32),
            pltpu.VMEM((NB2, DIM), jnp.float32),
            pltpu.VMEM_SHARED((NPAD, DIM), jnp.float32),
            pltpu.SemaphoreType.DMA,
            pltpu.SemaphoreType.DMA,
            pltpu.SemaphoreType.DMA,
            pltpu.SemaphoreType.DMA,
            pltpu.SemaphoreType.DMA,
            pltpu.SemaphoreType.DMA,
            pltpu.SemaphoreType.DMA,
            pltpu.SemaphoreType.DMA,
            pltpu.SemaphoreType.DMA,
            pltpu.SemaphoreType.DMA,
            pltpu.SemaphoreType.DMA,
            pltpu.SemaphoreType.DMA,
        ],
    )(src, dst2, R, xp, exf, zo)


def _final_body(op_ref, bias_ref, o_ref):
    o_ref[...] = (op_ref[0] + op_ref[1]) * (1.0 / HEADS) + bias_ref[...]


def _final(opart, bias):
    nblk = 10
    B = N_NODES // nblk
    return pl.pallas_call(
        _final_body,
        grid=(nblk,),
        in_specs=[
            pl.BlockSpec((2, B, DIM), lambda i: (0, i, 0)),
            pl.BlockSpec((1, DIM), lambda i: (0, 0)),
        ],
        out_specs=pl.BlockSpec((B, DIM), lambda i: (i, 0)),
        out_shape=jax.ShapeDtypeStruct((N_NODES, DIM), jnp.float32),
    )(opart, bias.reshape(1, DIM))


def kernel(x, edge_index, W, att_src, att_dst, bias):
    N = N_NODES
    E2 = edge_index.shape[1] + N

    loop = jnp.arange(N, dtype=jnp.int32)
    src = jnp.concatenate([edge_index[0].astype(jnp.int32), loop,
                           jnp.zeros((EP - E2,), jnp.int32)])
    dst = jnp.concatenate([edge_index[1].astype(jnp.int32), loop,
                           jnp.full((EP - E2,), N, jnp.int32)])
    src1 = src.reshape(EP // NB1, NB1)
    dst1 = dst.reshape(EP // NB1, NB1)
    dst2 = dst.reshape(EP // NB2, NB2)

    att_cat = jnp.concatenate([att_src, att_dst], axis=0)
    xp, acat, amaxo = _project(x, W, att_cat)

    pad = ((0, NPAD - N), (0, 0))
    S = jnp.pad(jnp.concatenate([acat[:, :8], acat[:, :8]], axis=1), pad)
    Dd = jnp.pad(jnp.concatenate([acat[:, 8:], acat[:, 8:]], axis=1), pad)
    amax16 = jnp.concatenate([amaxo[0, :8], amaxo[0, :8]])

    zd = jnp.zeros((NPAD, 16), jnp.float32)
    denom, exf = _sc_denom(src1, dst1, S, Dd, amax16, zd)
    R = _recip(denom)

    xsw = (xp.reshape(N, HEADS, 4, 2, 16).transpose(0, 1, 2, 4, 3)
           .reshape(N, HEADS * DIM).astype(jnp.bfloat16))

    zo = jnp.zeros((NPAD, DIM), jnp.float32)
    opart = _sc_msgs(src, dst2, R, xsw, exf, zo)
    return _final(opart, bias)
